# baseline (speedup 1.0000x reference)
import functools

import jax
import jax.numpy as jnp
from jax import lax
from jax.experimental import pallas as pl
from jax.experimental.pallas import tpu as pltpu
from jax.experimental.pallas import tpu_sc as plsc

DIS_RATIO = 3.0

_ATAN_COEFFS = (
    0.99999999, -0.33333138, 0.19993694, -0.14211105, 0.10667484,
    -0.07556891, 0.04327812, -0.01641311, 0.00293274,
)
_PI = 3.14159265358979
_HALF_PI = 1.57079632679490


def _atan_pos(t):
    inv = t > 1.0
    z = jnp.where(inv, 1.0 / jnp.maximum(t, 1e-30), t)
    zz = z * z
    p = jnp.float32(_ATAN_COEFFS[-1])
    for c in _ATAN_COEFFS[-2::-1]:
        p = p * zz + jnp.float32(c)
    r = z * p
    return jnp.where(inv, _HALF_PI - r, r)


def _atan2_abs(y, x):
    ax = jnp.abs(x)
    r = _atan_pos(y / ax)
    r = jnp.where(x < 0.0, _PI - r, r)
    return jnp.where((y == 0.0) & (x == 0.0), 0.0, r)


def _roll_p(a, k):
    return jnp.concatenate([a[:, -k:], a[:, :-k]], axis=1)


def _roll_m(a, k):
    return jnp.concatenate([a[:, k:], a[:, :k]], axis=1)


def _direct_change(cx, cy):
    total = None
    for k in (1, 3):
        v1x = cx - _roll_p(cx, k)
        v1y = cy - _roll_p(cy, k)
        v2x = _roll_m(cx, k) - cx
        v2y = _roll_m(cy, k) - cy
        cross = jnp.abs(v1x * v2y - v1y * v2x)
        dot = v1x * v2x + v1y * v2y
        d = _atan2_abs(cross, dot)
        total = d if total is None else total + d
    return total * 0.5


def _tc_kernel(vx_ref, vy_ref, cxr_ref, cyr_ref, mask_ref, par_ref,
               base_ref, wsel_ref, idx_ref, dtab_ref, cx_ref, cy_ref):
    cl = cxr_ref.shape[2]

    dtab_ref[0] = _direct_change(cxr_ref[0], cyr_ref[0])
    cx_ref[...] = jnp.transpose(cxr_ref[0], (1, 0))
    cy_ref[...] = jnp.transpose(cyr_ref[0], (1, 0))

    cx = cx_ref[...]
    cy = cy_ref[...]
    vx = vx_ref[0]
    vy = vy_ref[0]
    dx = cx - vx
    dy = cy - vy
    d2 = dx * dx + dy * dy
    m = jnp.min(d2, axis=0, keepdims=True)
    srow = jax.lax.broadcasted_iota(jnp.int32, (cl, 1), 0).astype(jnp.float32)
    idxf = jnp.min(jnp.where(d2 == m, srow, jnp.float32(cl)), axis=0,
                   keepdims=True)
    idx_ref[0] = idxf.astype(jnp.int32) + pl.program_id(0) * cl

    vc = jnp.sqrt(m + 1e-12) * jnp.float32(1.0 / DIS_RATIO)
    w0 = par_ref[0, 0]
    w1 = par_ref[0, 1]
    b0 = par_ref[0, 2]
    mask = mask_ref[0]
    base_ref[0] = vc * w0 * mask + b0
    wsel_ref[0] = w1 * mask


def _make_sc_gather(total, tab_len, nworkers, lanes):
    chunk = total // nworkers
    mesh = plsc.VectorSubcoreMesh(core_axis_name="c", subcore_axis_name="s")

    @functools.partial(
        pl.kernel, mesh=mesh,
        out_type=jax.ShapeDtypeStruct((total,), jnp.float32),
        scratch_types=[
            pltpu.VMEM((chunk,), jnp.int32),
            pltpu.VMEM((chunk,), jnp.float32),
            pltpu.VMEM((chunk,), jnp.float32),
            pltpu.VMEM((chunk,), jnp.float32),
            pltpu.VMEM((chunk,), jnp.float32),
            pltpu.SemaphoreType.DMA,
        ],
    )
    def sc_gather(dtab_hbm, idx_hbm, base_hbm, wsel_hbm, out_hbm,
                  idx_v, gath_v, base_v, wsel_v, out_v, sem):
        ncores = 2
        wid = lax.axis_index("s") * ncores + lax.axis_index("c")
        start = wid * chunk
        pltpu.sync_copy(idx_hbm.at[pl.ds(start, chunk)], idx_v)
        pltpu.async_copy(dtab_hbm.at[idx_v], gath_v, sem).wait()
        pltpu.sync_copy(base_hbm.at[pl.ds(start, chunk)], base_v)
        pltpu.sync_copy(wsel_hbm.at[pl.ds(start, chunk)], wsel_v)

        def body(i, carry):
            sl = pl.ds(i * lanes, lanes)
            out_v[sl] = base_v[sl] + gath_v[sl] * wsel_v[sl]
            return carry

        lax.fori_loop(0, chunk // lanes, body, 0)
        pltpu.sync_copy(out_v, out_hbm.at[pl.ds(start, chunk)])

    return sc_gather


@jax.jit
def kernel(vertices, valid_mask, contour, seg_logit, W, b):
    del seg_logit
    bsz, n, _ = vertices.shape
    cl = contour.shape[1]
    vx = vertices[..., 0].reshape(bsz, 1, n)
    vy = vertices[..., 1].reshape(bsz, 1, n)
    cxr = contour[..., 0].reshape(bsz, 1, cl)
    cyr = contour[..., 1].reshape(bsz, 1, cl)
    mask = valid_mask.reshape(bsz, 1, n)
    params = jnp.concatenate([W.reshape(-1), b.reshape(-1)]).reshape(1, 3)

    base, wsel, idx, dtab = pl.pallas_call(
        _tc_kernel,
        grid=(bsz,),
        in_specs=[
            pl.BlockSpec((1, 1, n), lambda i: (i, 0, 0)),
            pl.BlockSpec((1, 1, n), lambda i: (i, 0, 0)),
            pl.BlockSpec((1, 1, cl), lambda i: (i, 0, 0)),
            pl.BlockSpec((1, 1, cl), lambda i: (i, 0, 0)),
            pl.BlockSpec((1, 1, n), lambda i: (i, 0, 0)),
            pl.BlockSpec(memory_space=pltpu.SMEM),
        ],
        out_specs=[
            pl.BlockSpec((1, 1, n), lambda i: (i, 0, 0)),
            pl.BlockSpec((1, 1, n), lambda i: (i, 0, 0)),
            pl.BlockSpec((1, 1, n), lambda i: (i, 0, 0)),
            pl.BlockSpec((1, 1, cl), lambda i: (i, 0, 0)),
        ],
        out_shape=[
            jax.ShapeDtypeStruct((bsz, 1, n), jnp.float32),
            jax.ShapeDtypeStruct((bsz, 1, n), jnp.float32),
            jax.ShapeDtypeStruct((bsz, 1, n), jnp.int32),
            jax.ShapeDtypeStruct((bsz, 1, cl), jnp.float32),
        ],
        scratch_shapes=[
            pltpu.VMEM((cl, 1), jnp.float32),
            pltpu.VMEM((cl, 1), jnp.float32),
        ],
        compiler_params=pltpu.CompilerParams(
            dimension_semantics=("arbitrary",),
        ),
    )(vx, vy, cxr, cyr, mask, params)

    total = bsz * n
    sc_gather = _make_sc_gather(total, bsz * cl, 32, 16)
    out = sc_gather(dtab.reshape(bsz * cl), idx.reshape(total),
                    base.reshape(total), wsel.reshape(total))
    return out.reshape(bsz, n)
